# speedup vs baseline: 2.9619x; 2.9619x over previous
import functools

import jax
import jax.numpy as jnp
from jax import lax
from jax.experimental import pallas as pl
from jax.experimental.pallas import tpu as pltpu
from jax.experimental.pallas import tpu_sc as plsc

_D = 128
_CHUNK = 128


@functools.lru_cache(maxsize=None)
def _make_gather(B: int):
    info = plsc.get_sparse_core_info()
    nw = info.num_cores * info.num_subcores
    b_per_w = B // nw
    n_chunks = b_per_w // _CHUNK
    mesh = plsc.VectorSubcoreMesh(core_axis_name="c", subcore_axis_name="s")

    @functools.partial(
        pl.kernel,
        mesh=mesh,
        out_type=jax.ShapeDtypeStruct((B, _D), jnp.float32),
        scratch_types=[
            pltpu.VMEM((n_chunks, _CHUNK), jnp.int32),
            pltpu.VMEM((_CHUNK, _D), jnp.float32),
            pltpu.SemaphoreType.DMA,
        ],
    )
    def gather(idx_hbm, table_hbm, out_hbm, idx_v, rows_v, sem):
        wid = lax.axis_index("s") * info.num_cores + lax.axis_index("c")
        pltpu.sync_copy(idx_hbm.at[wid], idx_v)

        def body(j, carry):
            pltpu.async_copy(table_hbm.at[idx_v.at[j]], rows_v, sem).wait()
            pltpu.sync_copy(
                rows_v, out_hbm.at[pl.ds(wid * b_per_w + j * _CHUNK, _CHUNK)]
            )
            return carry

        lax.fori_loop(0, n_chunks, body, 0)

    return gather, nw, n_chunks


def kernel(token_ids, weight):
    S, T = token_ids.shape
    B = S * T
    gather, nw, n_chunks = _make_gather(B)
    idx = token_ids.reshape(nw, n_chunks, _CHUNK).astype(jnp.int32)
    out = gather(idx, weight)
    return out.reshape(S, T, _D)


# speedup vs baseline: 3.3067x; 1.1164x over previous
import functools

import jax
import jax.numpy as jnp
from jax import lax
from jax.experimental import pallas as pl
from jax.experimental.pallas import tpu as pltpu
from jax.experimental.pallas import tpu_sc as plsc

_D = 128
_CHUNK = 128
_K = 5


@functools.lru_cache(maxsize=None)
def _make_gather(B: int):
    info = plsc.get_sparse_core_info()
    nw = info.num_cores * info.num_subcores
    b_per_w = B // nw
    n_chunks = b_per_w // _CHUNK
    n_groups = n_chunks // _K
    mesh = plsc.VectorSubcoreMesh(core_axis_name="c", subcore_axis_name="s")

    @functools.partial(
        pl.kernel,
        mesh=mesh,
        out_type=jax.ShapeDtypeStruct((B, _D), jnp.float32),
        scratch_types=[
            pltpu.VMEM((n_chunks, _CHUNK), jnp.int32),
        ]
        + [pltpu.VMEM((_CHUNK, _D), jnp.float32) for _ in range(_K)]
        + [pltpu.SemaphoreType.DMA for _ in range(2 * _K)],
    )
    def gather(idx_hbm, table_hbm, out_hbm, idx_v, *bufs_and_sems):
        bufs = bufs_and_sems[:_K]
        gsem = bufs_and_sems[_K : 2 * _K]
        ssem = bufs_and_sems[2 * _K :]
        wid = lax.axis_index("s") * info.num_cores + lax.axis_index("c")
        base = wid * b_per_w
        pltpu.sync_copy(idx_hbm.at[wid], idx_v)

        def start_gather(b, j):
            pltpu.async_copy(table_hbm.at[idx_v.at[j]], bufs[b], gsem[b])

        def wait_gather(b, j):
            pltpu.make_async_copy(table_hbm.at[idx_v.at[j]], bufs[b], gsem[b]).wait()

        def start_store(b, j):
            pltpu.async_copy(
                bufs[b], out_hbm.at[pl.ds(base + j * _CHUNK, _CHUNK)], ssem[b]
            )

        def wait_store(b, j):
            pltpu.make_async_copy(
                bufs[b], out_hbm.at[pl.ds(base + j * _CHUNK, _CHUNK)], ssem[b]
            ).wait()

        for b in range(_K):
            start_gather(b, b)

        def body(g, carry):
            j0 = g * _K
            for b in range(_K):
                wait_gather(b, j0 + b)
                start_store(b, j0 + b)

            @pl.when(g < n_groups - 1)
            def _prefetch():
                for b in range(_K):
                    wait_store(b, j0 + b)
                    start_gather(b, j0 + _K + b)

            return carry

        lax.fori_loop(0, n_groups, body, 0)
        for b in range(_K):
            wait_store(b, (n_groups - 1) * _K + b)

    return gather, nw, n_chunks


def kernel(token_ids, weight):
    S, T = token_ids.shape
    B = S * T
    gather, nw, n_chunks = _make_gather(B)
    idx = token_ids.reshape(nw, n_chunks, _CHUNK).astype(jnp.int32)
    out = gather(idx, weight)
    return out.reshape(S, T, _D)


# speedup vs baseline: 5.9164x; 1.7892x over previous
import functools

import jax
import jax.numpy as jnp
from jax import lax
from jax.experimental import pallas as pl
from jax.experimental.pallas import tpu as pltpu
from jax.experimental.pallas import tpu_sc as plsc

_K = 8


@functools.lru_cache(maxsize=None)
def _make_gather(S: int, T: int, D: int):
    info = plsc.get_sparse_core_info()
    nw = info.num_cores * info.num_subcores
    rows_per_w = S // nw
    n_groups = rows_per_w // _K
    mesh = plsc.VectorSubcoreMesh(core_axis_name="c", subcore_axis_name="s")

    @functools.partial(
        pl.kernel,
        mesh=mesh,
        out_type=jax.ShapeDtypeStruct((S, T, D), jnp.float32),
        scratch_types=[
            pltpu.VMEM((rows_per_w, T), jnp.int32),
        ]
        + [pltpu.VMEM((T, D), jnp.float32) for _ in range(_K)]
        + [pltpu.SemaphoreType.DMA for _ in range(2 * _K)],
    )
    def gather(idx_hbm, table_hbm, out_hbm, idx_v, *bufs_and_sems):
        bufs = bufs_and_sems[:_K]
        gsem = bufs_and_sems[_K : 2 * _K]
        ssem = bufs_and_sems[2 * _K :]
        wid = lax.axis_index("s") * info.num_cores + lax.axis_index("c")
        base = wid * rows_per_w
        pltpu.sync_copy(idx_hbm.at[wid], idx_v)

        def start_gather(b, j):
            pltpu.async_copy(table_hbm.at[idx_v.at[j]], bufs[b], gsem[b])

        def wait_gather(b, j):
            pltpu.make_async_copy(table_hbm.at[idx_v.at[j]], bufs[b], gsem[b]).wait()

        def start_store(b, j):
            pltpu.async_copy(bufs[b], out_hbm.at[base + j], ssem[b])

        def wait_store(b, j):
            pltpu.make_async_copy(bufs[b], out_hbm.at[base + j], ssem[b]).wait()

        for b in range(_K):
            start_gather(b, b)

        def body(g, carry):
            j0 = g * _K
            for b in range(_K):
                wait_gather(b, j0 + b)
                start_store(b, j0 + b)

            @pl.when(g < n_groups - 1)
            def _prefetch():
                for b in range(_K):
                    wait_store(b, j0 + b)
                    start_gather(b, j0 + _K + b)

            return carry

        lax.fori_loop(0, n_groups, body, 0)
        for b in range(_K):
            wait_store(b, (n_groups - 1) * _K + b)

    return gather, nw


def kernel(token_ids, weight):
    S, T = token_ids.shape
    D = weight.shape[1]
    gather, nw = _make_gather(S, T, D)
    idx = token_ids.reshape(nw, S // nw, T).astype(jnp.int32)
    return gather(idx, weight)


# speedup vs baseline: 5.9383x; 1.0037x over previous
import functools

import jax
import jax.numpy as jnp
from jax import lax
from jax.experimental import pallas as pl
from jax.experimental.pallas import tpu as pltpu
from jax.experimental.pallas import tpu_sc as plsc

_K = 8


@functools.lru_cache(maxsize=None)
def _make_gather(S: int, T: int, D: int):
    info = plsc.get_sparse_core_info()
    nw = info.num_cores * info.num_subcores
    rows_per_w = S // nw
    n_groups = rows_per_w // _K
    mesh = plsc.VectorSubcoreMesh(core_axis_name="c", subcore_axis_name="s")

    @functools.partial(
        pl.kernel,
        mesh=mesh,
        out_type=jax.ShapeDtypeStruct((S, T, D), jnp.float32),
        scratch_types=[
            pltpu.VMEM((rows_per_w, T), jnp.int32),
        ]
        + [pltpu.VMEM((T, D), jnp.float32) for _ in range(_K)]
        + [pltpu.SemaphoreType.DMA for _ in range(2 * _K)],
        compiler_params=pltpu.CompilerParams(use_tc_tiling_on_sc=True),
    )
    def gather(idx_hbm, table_hbm, out_hbm, idx_v, *bufs_and_sems):
        bufs = bufs_and_sems[:_K]
        gsem = bufs_and_sems[_K : 2 * _K]
        ssem = bufs_and_sems[2 * _K :]
        wid = lax.axis_index("s") * info.num_cores + lax.axis_index("c")
        base = wid * rows_per_w
        pltpu.sync_copy(idx_hbm.at[wid], idx_v)

        def start_gather(b, j):
            pltpu.async_copy(table_hbm.at[idx_v.at[j]], bufs[b], gsem[b])

        def wait_gather(b, j):
            pltpu.make_async_copy(table_hbm.at[idx_v.at[j]], bufs[b], gsem[b]).wait()

        def start_store(b, j):
            pltpu.async_copy(bufs[b], out_hbm.at[base + j], ssem[b])

        def wait_store(b, j):
            pltpu.make_async_copy(bufs[b], out_hbm.at[base + j], ssem[b]).wait()

        for b in range(_K):
            start_gather(b, b)

        def body(g, carry):
            j0 = g * _K
            for b in range(_K):
                wait_gather(b, j0 + b)
                start_store(b, j0 + b)

            @pl.when(g < n_groups - 1)
            def _prefetch():
                for b in range(_K):
                    wait_store(b, j0 + b)
                    start_gather(b, j0 + _K + b)

            return carry

        lax.fori_loop(0, n_groups, body, 0)
        for b in range(_K):
            wait_store(b, (n_groups - 1) * _K + b)

    return gather, nw


def kernel(token_ids, weight):
    S, T = token_ids.shape
    D = weight.shape[1]
    gather, nw = _make_gather(S, T, D)
    idx = token_ids.reshape(nw, S // nw, T).astype(jnp.int32)
    return gather(idx, weight)


# speedup vs baseline: 10.1988x; 1.7175x over previous
import functools

import jax
import jax.numpy as jnp
from jax import lax
from jax.experimental import pallas as pl
from jax.experimental.pallas import tpu as pltpu
from jax.experimental.pallas import tpu_sc as plsc

_D = 128
_CHUNK = 128
_K = 5


@functools.lru_cache(maxsize=None)
def _make_gather(B: int):
    info = plsc.get_sparse_core_info()
    nw = info.num_cores * info.num_subcores
    b_per_w = B // nw
    n_chunks = b_per_w // _CHUNK
    n_groups = n_chunks // _K
    mesh = plsc.VectorSubcoreMesh(core_axis_name="c", subcore_axis_name="s")

    @functools.partial(
        pl.kernel,
        mesh=mesh,
        out_type=jax.ShapeDtypeStruct((B, _D), jnp.float32),
        scratch_types=[
            pltpu.VMEM((n_chunks, _CHUNK), jnp.int32),
        ]
        + [pltpu.VMEM((_CHUNK, _D), jnp.float32) for _ in range(_K)]
        + [pltpu.SemaphoreType.DMA for _ in range(2 * _K)],
    )
    def gather(idx_hbm, table_hbm, out_hbm, idx_v, *bufs_and_sems):
        bufs = bufs_and_sems[:_K]
        gsem = bufs_and_sems[_K : 2 * _K]
        ssem = bufs_and_sems[2 * _K :]
        wid = lax.axis_index("s") * info.num_cores + lax.axis_index("c")
        base = wid * b_per_w
        pltpu.sync_copy(idx_hbm.at[wid], idx_v)

        def start_gather(b, j):
            pltpu.async_copy(table_hbm.at[idx_v.at[j]], bufs[b], gsem[b])

        def wait_gather(b, j):
            pltpu.make_async_copy(table_hbm.at[idx_v.at[j]], bufs[b], gsem[b]).wait()

        def start_store(b, j):
            pltpu.async_copy(
                bufs[b], out_hbm.at[pl.ds(base + j * _CHUNK, _CHUNK)], ssem[b]
            )

        def wait_store(b, j):
            pltpu.make_async_copy(
                bufs[b], out_hbm.at[pl.ds(base + j * _CHUNK, _CHUNK)], ssem[b]
            ).wait()

        for b in range(_K):
            start_gather(b, b)

        def body(g, carry):
            j0 = g * _K
            for b in range(_K):
                wait_gather(b, j0 + b)
                start_store(b, j0 + b)

            @pl.when(g < n_groups - 1)
            def _prefetch():
                for b in range(_K):
                    wait_store(b, j0 + b)
                    start_gather(b, j0 + _K + b)

            return carry

        lax.fori_loop(0, n_groups, body, 0)
        for b in range(_K):
            wait_store(b, (n_groups - 1) * _K + b)

    return gather, nw, n_chunks


def kernel(token_ids, weight):
    S, T = token_ids.shape
    B = S * T
    gather, nw, n_chunks = _make_gather(B)
    idx = token_ids.T.reshape(nw, n_chunks, _CHUNK).astype(jnp.int32)
    out = gather(idx, weight)
    return out.reshape(T, S, _D).transpose(1, 0, 2)


# speedup vs baseline: 10.2398x; 1.0040x over previous
import functools

import jax
import jax.numpy as jnp
from jax import lax
from jax.experimental import pallas as pl
from jax.experimental.pallas import tpu as pltpu
from jax.experimental.pallas import tpu_sc as plsc

_D = 128
_CHUNK = 64
_K = 10


@functools.lru_cache(maxsize=None)
def _make_gather(B: int):
    info = plsc.get_sparse_core_info()
    nw = info.num_cores * info.num_subcores
    b_per_w = B // nw
    n_chunks = b_per_w // _CHUNK
    n_groups = n_chunks // _K
    mesh = plsc.VectorSubcoreMesh(core_axis_name="c", subcore_axis_name="s")

    @functools.partial(
        pl.kernel,
        mesh=mesh,
        out_type=jax.ShapeDtypeStruct((B, _D), jnp.float32),
        scratch_types=[
            pltpu.VMEM((n_chunks, _CHUNK), jnp.int32),
        ]
        + [pltpu.VMEM((_CHUNK, _D), jnp.float32) for _ in range(_K)]
        + [pltpu.SemaphoreType.DMA for _ in range(2 * _K)],
    )
    def gather(idx_hbm, table_hbm, out_hbm, idx_v, *bufs_and_sems):
        bufs = bufs_and_sems[:_K]
        gsem = bufs_and_sems[_K : 2 * _K]
        ssem = bufs_and_sems[2 * _K :]
        wid = lax.axis_index("s") * info.num_cores + lax.axis_index("c")
        base = wid * b_per_w
        pltpu.sync_copy(idx_hbm.at[wid], idx_v)

        def start_gather(b, j):
            pltpu.async_copy(table_hbm.at[idx_v.at[j]], bufs[b], gsem[b])

        def wait_gather(b, j):
            pltpu.make_async_copy(table_hbm.at[idx_v.at[j]], bufs[b], gsem[b]).wait()

        def start_store(b, j):
            pltpu.async_copy(
                bufs[b], out_hbm.at[pl.ds(base + j * _CHUNK, _CHUNK)], ssem[b]
            )

        def wait_store(b, j):
            pltpu.make_async_copy(
                bufs[b], out_hbm.at[pl.ds(base + j * _CHUNK, _CHUNK)], ssem[b]
            ).wait()

        for b in range(_K):
            start_gather(b, b)

        def body(g, carry):
            j0 = g * _K
            for b in range(_K):
                wait_gather(b, j0 + b)
                start_store(b, j0 + b)

            @pl.when(g < n_groups - 1)
            def _prefetch():
                for b in range(_K):
                    wait_store(b, j0 + b)
                    start_gather(b, j0 + _K + b)

            return carry

        lax.fori_loop(0, n_groups, body, 0)
        for b in range(_K):
            wait_store(b, (n_groups - 1) * _K + b)

    return gather, nw, n_chunks


def kernel(token_ids, weight):
    S, T = token_ids.shape
    B = S * T
    gather, nw, n_chunks = _make_gather(B)
    idx = token_ids.T.reshape(nw, n_chunks, _CHUNK).astype(jnp.int32)
    out = gather(idx, weight)
    return out.reshape(T, S, _D).transpose(1, 0, 2)


# speedup vs baseline: 10.4598x; 1.0215x over previous
import functools

import jax
import jax.numpy as jnp
from jax import lax
from jax.experimental import pallas as pl
from jax.experimental.pallas import tpu as pltpu
from jax.experimental.pallas import tpu_sc as plsc

_D = 128
_CHUNK = 64
_K = 10


@functools.lru_cache(maxsize=None)
def _make_gather(B: int):
    info = plsc.get_sparse_core_info()
    nw = info.num_cores * info.num_subcores
    b_per_w = B // nw
    n_chunks = b_per_w // _CHUNK
    n_groups = n_chunks // _K
    mesh = plsc.VectorSubcoreMesh(core_axis_name="c", subcore_axis_name="s")

    @functools.partial(
        pl.kernel,
        mesh=mesh,
        out_type=jax.ShapeDtypeStruct((B, _D), jnp.float32),
        scratch_types=[
            pltpu.VMEM((n_chunks, _CHUNK), jnp.int32),
        ]
        + [pltpu.VMEM((_CHUNK, _D), jnp.float32) for _ in range(_K)]
        + [pltpu.SemaphoreType.DMA for _ in range(2 * _K)],
    )
    def gather(idx_hbm, table_hbm, out_hbm, idx_v, *bufs_and_sems):
        bufs = bufs_and_sems[:_K]
        gsem = bufs_and_sems[_K : 2 * _K]
        ssem = bufs_and_sems[2 * _K :]
        wid = lax.axis_index("s") * info.num_cores + lax.axis_index("c")
        base = wid * b_per_w
        pltpu.sync_copy(idx_hbm.at[wid], idx_v)

        def start_gather(b, j):
            pltpu.async_copy(table_hbm.at[idx_v.at[j]], bufs[b], gsem[b])

        def wait_gather(b, j):
            pltpu.make_async_copy(table_hbm.at[idx_v.at[j]], bufs[b], gsem[b]).wait()

        def start_store(b, j):
            pltpu.async_copy(
                bufs[b], out_hbm.at[pl.ds(base + j * _CHUNK, _CHUNK)], ssem[b]
            )

        def wait_store(b, j):
            pltpu.make_async_copy(
                bufs[b], out_hbm.at[pl.ds(base + j * _CHUNK, _CHUNK)], ssem[b]
            ).wait()

        for c in range(_K - 2):
            start_gather(c, c)
        for b in range(_K):
            wait_gather(b, b)
            start_store(b, b)
            c = b + _K - 2
            if c < n_chunks:
                if b >= 2:
                    wait_store((b - 2) % _K, b - 2)
                start_gather((b - 2) % _K, c)

        def body(g, carry):
            j0 = g * _K
            for b in range(_K):
                j = j0 + b
                wait_gather(b, j)
                start_store(b, j)
                bc = (b - 2) % _K

                @pl.when(j + _K - 2 < n_chunks)
                def _prefetch():
                    wait_store(bc, j - 2)
                    start_gather(bc, j + _K - 2)

            return carry

        lax.fori_loop(1, n_groups, body, 0)
        for b in range(_K):
            wait_store(b, (n_groups - 1) * _K + b)

    return gather, nw, n_chunks


def kernel(token_ids, weight):
    S, T = token_ids.shape
    B = S * T
    gather, nw, n_chunks = _make_gather(B)
    idx = token_ids.T.reshape(nw, n_chunks, _CHUNK).astype(jnp.int32)
    out = gather(idx, weight)
    return out.reshape(T, S, _D).transpose(1, 0, 2)


# speedup vs baseline: 10.4796x; 1.0019x over previous
import functools

import jax
import jax.numpy as jnp
from jax import lax
from jax.experimental import pallas as pl
from jax.experimental.pallas import tpu as pltpu
from jax.experimental.pallas import tpu_sc as plsc

_D = 128
_CHUNK = 64
_K = 10


@functools.lru_cache(maxsize=None)
def _make_gather(B: int):
    info = plsc.get_sparse_core_info()
    nw = info.num_cores * info.num_subcores
    b_per_w = B // nw
    n_chunks = b_per_w // _CHUNK
    n_groups = n_chunks // _K
    mesh = plsc.VectorSubcoreMesh(core_axis_name="c", subcore_axis_name="s")

    @functools.partial(
        pl.kernel,
        mesh=mesh,
        out_type=jax.ShapeDtypeStruct((B, _D), jnp.float32),
        scratch_types=[
            pltpu.VMEM((n_chunks, _CHUNK), jnp.int32),
        ]
        + [pltpu.VMEM((_CHUNK, _D), jnp.float32) for _ in range(_K)]
        + [pltpu.SemaphoreType.DMA for _ in range(2 * _K)],
        compiler_params=pltpu.CompilerParams(skip_device_barrier=True),
    )
    def gather(idx_hbm, table_hbm, out_hbm, idx_v, *bufs_and_sems):
        bufs = bufs_and_sems[:_K]
        gsem = bufs_and_sems[_K : 2 * _K]
        ssem = bufs_and_sems[2 * _K :]
        wid = lax.axis_index("s") * info.num_cores + lax.axis_index("c")
        base = wid * b_per_w
        pltpu.sync_copy(idx_hbm.at[wid], idx_v)

        def start_gather(b, j):
            pltpu.async_copy(table_hbm.at[idx_v.at[j]], bufs[b], gsem[b])

        def wait_gather(b, j):
            pltpu.make_async_copy(table_hbm.at[idx_v.at[j]], bufs[b], gsem[b]).wait()

        def start_store(b, j):
            pltpu.async_copy(
                bufs[b], out_hbm.at[pl.ds(base + j * _CHUNK, _CHUNK)], ssem[b]
            )

        def wait_store(b, j):
            pltpu.make_async_copy(
                bufs[b], out_hbm.at[pl.ds(base + j * _CHUNK, _CHUNK)], ssem[b]
            ).wait()

        for c in range(_K - 2):
            start_gather(c, c)
        for b in range(_K):
            wait_gather(b, b)
            start_store(b, b)
            c = b + _K - 2
            if c < n_chunks:
                if b >= 2:
                    wait_store((b - 2) % _K, b - 2)
                start_gather((b - 2) % _K, c)

        def body(g, carry):
            j0 = g * _K
            for b in range(_K):
                j = j0 + b
                wait_gather(b, j)
                start_store(b, j)
                bc = (b - 2) % _K

                @pl.when(j + _K - 2 < n_chunks)
                def _prefetch():
                    wait_store(bc, j - 2)
                    start_gather(bc, j + _K - 2)

            return carry

        lax.fori_loop(1, n_groups, body, 0)
        for b in range(_K):
            wait_store(b, (n_groups - 1) * _K + b)

    return gather, nw, n_chunks


def kernel(token_ids, weight):
    S, T = token_ids.shape
    B = S * T
    gather, nw, n_chunks = _make_gather(B)
    idx = token_ids.T.reshape(nw, n_chunks, _CHUNK).astype(jnp.int32)
    out = gather(idx, weight)
    return out.reshape(T, S, _D).transpose(1, 0, 2)
